# baseline (speedup 1.0000x reference)
import functools

import jax
import jax.numpy as jnp
from jax import lax
from jax.experimental import pallas as pl
from jax.experimental.pallas import tpu as pltpu
from jax.experimental.pallas import tpu_sc as plsc

D_MODEL = 1024
LANES = 16
NUM_CORES = 2
NUM_SUBCORES = 16
NUM_WORKERS = NUM_CORES * NUM_SUBCORES
CHUNK = 16


def _embed_body(n_rows, seq_len, table_hbm, pos_hbm, idx_hbm, out_hbm,
                idx_v, tok_v, pos_v, sem):
    rows_per_w = n_rows // NUM_WORKERS
    wid = lax.axis_index("s") * NUM_CORES + lax.axis_index("c")
    base = wid * rows_per_w
    pos_base = lax.rem(base, seq_len)

    pltpu.sync_copy(idx_hbm.at[pl.ds(base, rows_per_w)], idx_v)

    @pl.loop(0, rows_per_w // CHUNK)
    def _chunk(g):
        off = g * CHUNK
        gather = pltpu.async_copy(
            table_hbm.at[idx_v.at[pl.ds(off, CHUNK)]], tok_v, sem)
        pltpu.sync_copy(pos_hbm.at[pl.ds(pos_base + off, CHUNK)], pos_v)
        gather.wait()
        for r in range(CHUNK):
            for j in range(D_MODEL // LANES):
                sl = pl.ds(j * LANES, LANES)
                plsc.addupdate(pos_v.at[r, sl], tok_v[r, sl])
        pltpu.sync_copy(pos_v, out_hbm.at[pl.ds(base + off, CHUNK)])


def kernel(x, token_table, pos_table):
    batch, seq_len = x.shape
    n_rows = batch * seq_len
    d = token_table.shape[1]
    xf = x.reshape(n_rows).astype(jnp.int32)

    mesh = plsc.VectorSubcoreMesh(core_axis_name="c", subcore_axis_name="s")
    run = pl.kernel(
        functools.partial(_embed_body, n_rows, seq_len),
        out_type=jax.ShapeDtypeStruct((n_rows, d), jnp.float32),
        mesh=mesh,
        scratch_types=[
            pltpu.VMEM((n_rows // NUM_WORKERS,), jnp.int32),
            pltpu.VMEM((CHUNK, d), jnp.float32),
            pltpu.VMEM((CHUNK, d), jnp.float32),
            pltpu.SemaphoreType.DMA,
        ],
    )
    out = run(token_table, pos_table, xf)
    return out.reshape(batch, seq_len, d)


# speedup vs baseline: 2.9575x; 2.9575x over previous
import functools

import jax
import jax.numpy as jnp
from jax import lax
from jax.experimental import pallas as pl
from jax.experimental.pallas import tpu as pltpu
from jax.experimental.pallas import tpu_sc as plsc

D_MODEL = 1024
LANES = 16
VECS = D_MODEL // LANES
NUM_CORES = 2
NUM_SUBCORES = 16
NUM_WORKERS = NUM_CORES * NUM_SUBCORES
CHUNK = 16
NBATCH = 4
ILV = 8


def _embed_body(seq_len, table_hbm, pos_hbm, idx_hbm, out_hbm,
                idx_v, tok_v, pos_v,
                gsem0, gsem1, gsem2, gsem3,
                ssem0, ssem1, ssem2, ssem3,
                psem0, psem1):
    gsems = (gsem0, gsem1, gsem2, gsem3)
    ssems = (ssem0, ssem1, ssem2, ssem3)
    psems = (psem0, psem1)

    span = seq_len // NUM_WORKERS
    n_pchunks = span // CHUNK
    wid = lax.axis_index("s") * NUM_CORES + lax.axis_index("c")
    s0 = wid * span

    for bi in range(NBATCH):
        pltpu.sync_copy(idx_hbm.at[pl.ds(bi * seq_len + s0, span)],
                        idx_v.at[bi])

    def issue_gather(p, bi):
        pltpu.async_copy(
            table_hbm.at[idx_v.at[bi, pl.ds(p * CHUNK, CHUNK)]],
            tok_v.at[bi], gsems[bi])

    def wait_gather(bi):
        pltpu.make_async_copy(
            table_hbm.at[idx_v.at[bi, pl.ds(0, CHUNK)]],
            tok_v.at[bi], gsems[bi]).wait()

    def issue_store(p, bi):
        pltpu.async_copy(
            tok_v.at[bi],
            out_hbm.at[pl.ds(bi * seq_len + s0 + p * CHUNK, CHUNK)],
            ssems[bi])

    def wait_store(bi):
        pltpu.make_async_copy(
            tok_v.at[bi],
            out_hbm.at[pl.ds(bi * seq_len + s0, CHUNK)],
            ssems[bi]).wait()

    def issue_pos(p, q):
        pltpu.async_copy(
            pos_hbm.at[pl.ds(s0 + p * CHUNK, CHUNK)], pos_v.at[q], psems[q])

    def wait_pos(q):
        pltpu.make_async_copy(
            pos_hbm.at[pl.ds(s0, CHUNK)], pos_v.at[q], psems[q]).wait()

    issue_pos(0, 0)
    for bi in range(NBATCH):
        issue_gather(0, bi)

    @pl.loop(0, n_pchunks, step=2)
    def _pchunk(p0):
        for dp in range(2):
            p = p0 + dp
            q = dp
            wait_pos(q)
            if dp == 0:
                issue_pos(p + 1, 1 - q)
            else:
                @pl.when(p0 + 2 < n_pchunks)
                def _():
                    issue_pos(p + 1, 1 - q)

            for bi in range(NBATCH):
                wait_gather(bi)

                @pl.loop(0, CHUNK)
                def _row(r):
                    for j0 in range(0, VECS, ILV):
                        vs = [pos_v[q, r, pl.ds((j0 + k) * LANES, LANES)]
                              for k in range(ILV)]
                        for k in range(ILV):
                            plsc.addupdate(
                                tok_v.at[bi, r,
                                         pl.ds((j0 + k) * LANES, LANES)],
                                vs[k])

                issue_store(p, bi)

            if dp == 0:
                for bi in range(NBATCH):
                    wait_store(bi)
                    issue_gather(p + 1, bi)
            else:
                @pl.when(p0 + 2 < n_pchunks)
                def _():
                    for bi in range(NBATCH):
                        wait_store(bi)
                        issue_gather(p + 1, bi)

    for bi in range(NBATCH):
        wait_store(bi)


def kernel(x, token_table, pos_table):
    batch, seq_len = x.shape
    n_rows = batch * seq_len
    d = token_table.shape[1]
    xf = x.reshape(n_rows).astype(jnp.int32)
    span = seq_len // NUM_WORKERS

    mesh = plsc.VectorSubcoreMesh(core_axis_name="c", subcore_axis_name="s")
    run = pl.kernel(
        functools.partial(_embed_body, seq_len),
        out_type=jax.ShapeDtypeStruct((n_rows, d), jnp.float32),
        mesh=mesh,
        scratch_types=[
            pltpu.VMEM((NBATCH, span), jnp.int32),
            pltpu.VMEM((NBATCH, CHUNK, d), jnp.float32),
            pltpu.VMEM((2, CHUNK, d), jnp.float32),
        ] + [pltpu.SemaphoreType.DMA] * 10,
    )
    out = run(token_table, pos_table, xf)
    return out.reshape(batch, seq_len, d)


# speedup vs baseline: 3.5276x; 1.1928x over previous
import functools

import jax
import jax.numpy as jnp
from jax import lax
from jax.experimental import pallas as pl
from jax.experimental.pallas import tpu as pltpu
from jax.experimental.pallas import tpu_sc as plsc

D_MODEL = 1024
LANES = 16
VECS = D_MODEL // LANES
NUM_CORES = 2
NUM_SUBCORES = 16
NUM_WORKERS = NUM_CORES * NUM_SUBCORES
CHUNK = 8
NBATCH = 4
ILV = 8


def _embed_body(seq_len, table_hbm, pos_hbm, idx_hbm, out_hbm,
                idx_v, tok_v, pos_v, *sems):
    gsems = [sems[bi * 2:bi * 2 + 2] for bi in range(NBATCH)]
    ssems = [sems[8 + bi * 2:8 + bi * 2 + 2] for bi in range(NBATCH)]
    psems = sems[16:18]

    span = seq_len // NUM_WORKERS
    n_pchunks = span // CHUNK
    wid = lax.axis_index("s") * NUM_CORES + lax.axis_index("c")
    s0 = wid * span

    pltpu.sync_copy(idx_hbm.at[:, pl.ds(s0, span)], idx_v)

    def issue_gather(p, bi, e):
        pltpu.async_copy(
            table_hbm.at[idx_v.at[bi, pl.ds(p * CHUNK, CHUNK)]],
            tok_v.at[bi, e], gsems[bi][e])

    def wait_gather(bi, e):
        pltpu.make_async_copy(
            table_hbm.at[idx_v.at[bi, pl.ds(0, CHUNK)]],
            tok_v.at[bi, e], gsems[bi][e]).wait()

    def issue_store(p, bi, e):
        pltpu.async_copy(
            tok_v.at[bi, e],
            out_hbm.at[bi, pl.ds(s0 + p * CHUNK, CHUNK)], ssems[bi][e])

    def wait_store(bi, e):
        pltpu.make_async_copy(
            tok_v.at[bi, e],
            out_hbm.at[bi, pl.ds(s0, CHUNK)], ssems[bi][e]).wait()

    def issue_pos(p, q):
        pltpu.async_copy(
            pos_hbm.at[pl.ds(s0 + p * CHUNK, CHUNK)], pos_v.at[q], psems[q])

    def wait_pos(q):
        pltpu.make_async_copy(
            pos_hbm.at[pl.ds(s0, CHUNK)], pos_v.at[q], psems[q]).wait()

    issue_pos(0, 0)
    for bi in range(NBATCH):
        issue_gather(0, bi, 0)

    @pl.loop(0, n_pchunks, step=2)
    def _pchunk(p0):
        for dp in range(2):
            p = p0 + dp
            e = dp
            wait_pos(e)
            if dp == 0:
                issue_pos(p + 1, 1 - e)
            else:
                @pl.when(p0 + 2 < n_pchunks)
                def _():
                    issue_pos(p + 1, 1 - e)

            if dp == 0:
                @pl.when(p0 >= 1)
                def _():
                    for bi in range(NBATCH):
                        wait_store(bi, 1 - e)
                        issue_gather(p + 1, bi, 1 - e)

                @pl.when(p0 < 1)
                def _():
                    for bi in range(NBATCH):
                        issue_gather(p + 1, bi, 1 - e)
            else:
                @pl.when(p0 + 2 < n_pchunks)
                def _():
                    for bi in range(NBATCH):
                        wait_store(bi, 1 - e)
                        issue_gather(p + 1, bi, 1 - e)

            for bi in range(NBATCH):
                wait_gather(bi, e)

                @pl.loop(0, CHUNK)
                def _row(r):
                    for j0 in range(0, VECS, ILV):
                        vs = [pos_v[e, r, pl.ds((j0 + k) * LANES, LANES)]
                              for k in range(ILV)]
                        for k in range(ILV):
                            plsc.addupdate(
                                tok_v.at[bi, e, r,
                                         pl.ds((j0 + k) * LANES, LANES)],
                                vs[k])

                issue_store(p, bi, e)

    for bi in range(NBATCH):
        wait_store(bi, 0)
        wait_store(bi, 1)


def kernel(x, token_table, pos_table):
    batch, seq_len = x.shape
    d = token_table.shape[1]
    xi = x.astype(jnp.int32)
    span = seq_len // NUM_WORKERS

    mesh = plsc.VectorSubcoreMesh(core_axis_name="c", subcore_axis_name="s")
    run = pl.kernel(
        functools.partial(_embed_body, seq_len),
        out_type=jax.ShapeDtypeStruct((batch, seq_len, d), jnp.float32),
        mesh=mesh,
        scratch_types=[
            pltpu.VMEM((NBATCH, span), jnp.int32),
            pltpu.VMEM((NBATCH, 2, CHUNK, d), jnp.float32),
            pltpu.VMEM((2, CHUNK, d), jnp.float32),
        ] + [pltpu.SemaphoreType.DMA] * 18,
    )
    return run(token_table, pos_table, xi)


# speedup vs baseline: 3.6993x; 1.0487x over previous
import functools

import jax
import jax.numpy as jnp
from jax import lax
from jax.experimental import pallas as pl
from jax.experimental.pallas import tpu as pltpu
from jax.experimental.pallas import tpu_sc as plsc

D_MODEL = 1024
LANES = 16
VECS = D_MODEL // LANES
NUM_CORES = 2
NUM_SUBCORES = 16
NUM_WORKERS = NUM_CORES * NUM_SUBCORES
CHUNK = 8
NBATCH = 4
ILV = 8


def _embed_body(seq_len, table_hbm, pos_hbm, idx_hbm, out_hbm,
                idx_v, tok_v, pos_v, *sems):
    gsems = [sems[bi * 2:bi * 2 + 2] for bi in range(NBATCH)]
    ssems = [sems[8 + bi * 2:8 + bi * 2 + 2] for bi in range(NBATCH)]
    psems = sems[16:18]

    span = seq_len // NUM_WORKERS
    n_pchunks = span // CHUNK
    wid = lax.axis_index("s") * NUM_CORES + lax.axis_index("c")
    s0 = wid * span

    pltpu.sync_copy(idx_hbm.at[:, pl.ds(s0, span)], idx_v)

    def issue_gather(p, bi, e):
        pltpu.async_copy(
            table_hbm.at[idx_v.at[bi, pl.ds(p * CHUNK, CHUNK)]],
            tok_v.at[bi, e], gsems[bi][e])

    def wait_gather(bi, e):
        pltpu.make_async_copy(
            table_hbm.at[idx_v.at[bi, pl.ds(0, CHUNK)]],
            tok_v.at[bi, e], gsems[bi][e]).wait()

    def issue_store(p, bi, e):
        pltpu.async_copy(
            tok_v.at[bi, e],
            out_hbm.at[bi, pl.ds(s0 + p * CHUNK, CHUNK)], ssems[bi][e])

    def wait_store(bi, e):
        pltpu.make_async_copy(
            tok_v.at[bi, e],
            out_hbm.at[bi, pl.ds(s0, CHUNK)], ssems[bi][e]).wait()

    def issue_pos(p, q):
        pltpu.async_copy(
            pos_hbm.at[pl.ds(s0 + p * CHUNK, CHUNK)], pos_v.at[q], psems[q])

    def wait_pos(q):
        pltpu.make_async_copy(
            pos_hbm.at[pl.ds(s0, CHUNK)], pos_v.at[q], psems[q]).wait()

    issue_pos(0, 0)
    for bi in range(NBATCH):
        issue_gather(0, bi, 0)

    @pl.loop(0, n_pchunks, step=2)
    def _pchunk(p0):
        for dp in range(2):
            p = p0 + dp
            e = dp
            wait_pos(e)
            if dp == 0:
                issue_pos(p + 1, 1 - e)
            else:
                @pl.when(p0 + 2 < n_pchunks)
                def _():
                    issue_pos(p + 1, 1 - e)

            if dp == 0:
                @pl.when(p0 >= 1)
                def _():
                    for bi in range(NBATCH):
                        wait_store(bi, 1 - e)
                        issue_gather(p + 1, bi, 1 - e)

                @pl.when(p0 < 1)
                def _():
                    for bi in range(NBATCH):
                        issue_gather(p + 1, bi, 1 - e)
            else:
                @pl.when(p0 + 2 < n_pchunks)
                def _():
                    for bi in range(NBATCH):
                        wait_store(bi, 1 - e)
                        issue_gather(p + 1, bi, 1 - e)

            for bi in range(NBATCH):
                wait_gather(bi, e)

            @pl.loop(0, CHUNK)
            def _row(r):
                for j0 in range(0, VECS, ILV):
                    vs = [pos_v[e, r, pl.ds((j0 + k) * LANES, LANES)]
                          for k in range(ILV)]
                    for bi in range(NBATCH):
                        for k in range(ILV):
                            plsc.addupdate(
                                tok_v.at[bi, e, r,
                                         pl.ds((j0 + k) * LANES, LANES)],
                                vs[k])

            for bi in range(NBATCH):
                issue_store(p, bi, e)

    for bi in range(NBATCH):
        wait_store(bi, 0)
        wait_store(bi, 1)


def kernel(x, token_table, pos_table):
    batch, seq_len = x.shape
    d = token_table.shape[1]
    xi = x.astype(jnp.int32)
    span = seq_len // NUM_WORKERS

    mesh = plsc.VectorSubcoreMesh(core_axis_name="c", subcore_axis_name="s")
    run = pl.kernel(
        functools.partial(_embed_body, seq_len),
        out_type=jax.ShapeDtypeStruct((batch, seq_len, d), jnp.float32),
        mesh=mesh,
        scratch_types=[
            pltpu.VMEM((NBATCH, span), jnp.int32),
            pltpu.VMEM((NBATCH, 2, CHUNK, d), jnp.float32),
            pltpu.VMEM((2, CHUNK, d), jnp.float32),
        ] + [pltpu.SemaphoreType.DMA] * 18,
    )
    return run(token_table, pos_table, xi)


# speedup vs baseline: 3.7418x; 1.0115x over previous
import functools

import jax
import jax.numpy as jnp
from jax import lax
from jax.experimental import pallas as pl
from jax.experimental.pallas import tpu as pltpu
from jax.experimental.pallas import tpu_sc as plsc

D_MODEL = 1024
LANES = 16
VECS = D_MODEL // LANES
NUM_CORES = 2
NUM_SUBCORES = 16
NUM_WORKERS = NUM_CORES * NUM_SUBCORES
CHUNK = 8
NBATCH = 4
NSLOT = 3
ILV = 8


def _embed_body(seq_len, table_hbm, pos_hbm, idx_hbm, out_hbm,
                idx_v, tok_v, pos_v, gsem, ssem, psem):
    span = seq_len // NUM_WORKERS
    n_pchunks = span // CHUNK
    wid = lax.axis_index("s") * NUM_CORES + lax.axis_index("c")
    s0 = wid * span

    pltpu.sync_copy(idx_hbm.at[:, pl.ds(s0, span)], idx_v)

    def issue_gather(p, bi, s):
        pltpu.async_copy(
            table_hbm.at[idx_v.at[bi, pl.ds(p * CHUNK, CHUNK)]],
            tok_v.at[bi, s], gsem.at[bi, s])

    def wait_gather(bi, s):
        pltpu.make_async_copy(
            table_hbm.at[idx_v.at[bi, pl.ds(0, CHUNK)]],
            tok_v.at[bi, s], gsem.at[bi, s]).wait()

    def issue_store(p, bi, s):
        pltpu.async_copy(
            tok_v.at[bi, s],
            out_hbm.at[bi, pl.ds(s0 + p * CHUNK, CHUNK)], ssem.at[bi, s])

    def wait_store(bi, s):
        pltpu.make_async_copy(
            tok_v.at[bi, s],
            out_hbm.at[bi, pl.ds(s0, CHUNK)], ssem.at[bi, s]).wait()

    def issue_pos(p, q):
        pltpu.async_copy(
            pos_hbm.at[pl.ds(s0 + p * CHUNK, CHUNK)], pos_v.at[q],
            psem.at[q])

    def wait_pos(q):
        pltpu.make_async_copy(
            pos_hbm.at[pl.ds(s0, CHUNK)], pos_v.at[q], psem.at[q]).wait()

    issue_pos(0, 0)
    for bi in range(NBATCH):
        issue_gather(0, bi, 0)

    @pl.loop(0, n_pchunks)
    def _pchunk(p):
        q = lax.rem(p, 2)
        s = lax.rem(p, NSLOT)
        sn = lax.rem(p + 1, NSLOT)
        wait_pos(q)

        @pl.when(p + 1 < n_pchunks)
        def _():
            issue_pos(p + 1, 1 - q)

            @pl.when(p >= NSLOT - 1)
            def _():
                for bi in range(NBATCH):
                    wait_store(bi, sn)
            for bi in range(NBATCH):
                issue_gather(p + 1, bi, sn)

        for bi in range(NBATCH):
            wait_gather(bi, s)

        @pl.loop(0, CHUNK)
        def _row(r):
            for j0 in range(0, VECS, ILV):
                vs = [pos_v[q, r, pl.ds((j0 + k) * LANES, LANES)]
                      for k in range(ILV)]
                for bi in range(NBATCH):
                    for k in range(ILV):
                        plsc.addupdate(
                            tok_v.at[bi, s, r,
                                     pl.ds((j0 + k) * LANES, LANES)],
                            vs[k])

        for bi in range(NBATCH):
            issue_store(p, bi, s)

    for bi in range(NBATCH):
        wait_store(bi, lax.rem(n_pchunks - 2, NSLOT))
        wait_store(bi, lax.rem(n_pchunks - 1, NSLOT))


def kernel(x, token_table, pos_table):
    batch, seq_len = x.shape
    d = token_table.shape[1]
    xi = x.astype(jnp.int32)
    span = seq_len // NUM_WORKERS

    mesh = plsc.VectorSubcoreMesh(core_axis_name="c", subcore_axis_name="s")
    run = pl.kernel(
        functools.partial(_embed_body, seq_len),
        out_type=jax.ShapeDtypeStruct((batch, seq_len, d), jnp.float32),
        mesh=mesh,
        scratch_types=[
            pltpu.VMEM((NBATCH, span), jnp.int32),
            pltpu.VMEM((NBATCH, NSLOT, CHUNK, d), jnp.float32),
            pltpu.VMEM((2, CHUNK, d), jnp.float32),
            pltpu.SemaphoreType.DMA((NBATCH, NSLOT)),
            pltpu.SemaphoreType.DMA((NBATCH, NSLOT)),
            pltpu.SemaphoreType.DMA((2,)),
        ],
    )
    return run(token_table, pos_table, xi)


# speedup vs baseline: 3.7575x; 1.0042x over previous
import functools

import jax
import jax.numpy as jnp
from jax import lax
from jax.experimental import pallas as pl
from jax.experimental.pallas import tpu as pltpu
from jax.experimental.pallas import tpu_sc as plsc

D_MODEL = 1024
LANES = 16
VECS = D_MODEL // LANES
NUM_CORES = 2
NUM_SUBCORES = 16
NUM_WORKERS = NUM_CORES * NUM_SUBCORES
CHUNK = 8
NBATCH = 4
NSLOT = 3
ILV = 16


def _embed_body(seq_len, table_hbm, pos_hbm, idx_hbm, out_hbm,
                idx_v, tok_v, pos_v, gsem, ssem, psem):
    span = seq_len // NUM_WORKERS
    n_pchunks = span // CHUNK
    wid = lax.axis_index("s") * NUM_CORES + lax.axis_index("c")
    s0 = wid * span

    pltpu.async_copy(pos_hbm.at[pl.ds(s0, CHUNK)], pos_v.at[0], psem.at[0])

    pltpu.sync_copy(idx_hbm.at[:, pl.ds(s0, span)], idx_v)

    def issue_gather(p, bi, s):
        pltpu.async_copy(
            table_hbm.at[idx_v.at[bi, pl.ds(p * CHUNK, CHUNK)]],
            tok_v.at[bi, s], gsem.at[bi, s])

    def wait_gather(bi, s):
        pltpu.make_async_copy(
            table_hbm.at[idx_v.at[bi, pl.ds(0, CHUNK)]],
            tok_v.at[bi, s], gsem.at[bi, s]).wait()

    def issue_store(p, bi, s):
        pltpu.async_copy(
            tok_v.at[bi, s],
            out_hbm.at[bi, pl.ds(s0 + p * CHUNK, CHUNK)], ssem.at[bi, s])

    def wait_store(bi, s):
        pltpu.make_async_copy(
            tok_v.at[bi, s],
            out_hbm.at[bi, pl.ds(s0, CHUNK)], ssem.at[bi, s]).wait()

    def issue_pos(p, q):
        pltpu.async_copy(
            pos_hbm.at[pl.ds(s0 + p * CHUNK, CHUNK)], pos_v.at[q],
            psem.at[q])

    def wait_pos(q):
        pltpu.make_async_copy(
            pos_hbm.at[pl.ds(s0, CHUNK)], pos_v.at[q], psem.at[q]).wait()

    for bi in range(NBATCH):
        issue_gather(0, bi, 0)

    @pl.loop(0, n_pchunks)
    def _pchunk(p):
        q = lax.rem(p, 2)
        s = lax.rem(p, NSLOT)
        sn = lax.rem(p + 1, NSLOT)
        wait_pos(q)

        @pl.when(p + 1 < n_pchunks)
        def _():
            issue_pos(p + 1, 1 - q)

            @pl.when(p >= NSLOT - 1)
            def _():
                for bi in range(NBATCH):
                    wait_store(bi, sn)
            for bi in range(NBATCH):
                issue_gather(p + 1, bi, sn)

        for bi in range(NBATCH):
            wait_gather(bi, s)

        @pl.loop(0, CHUNK)
        def _row(r):
            for j0 in range(0, VECS, ILV):
                vs = [pos_v[q, r, pl.ds((j0 + k) * LANES, LANES)]
                      for k in range(ILV)]
                for bi in range(NBATCH):
                    for k in range(ILV):
                        plsc.addupdate(
                            tok_v.at[bi, s, r,
                                     pl.ds((j0 + k) * LANES, LANES)],
                            vs[k])

        for bi in range(NBATCH):
            issue_store(p, bi, s)

    for bi in range(NBATCH):
        wait_store(bi, lax.rem(n_pchunks - 2, NSLOT))
        wait_store(bi, lax.rem(n_pchunks - 1, NSLOT))


def kernel(x, token_table, pos_table):
    batch, seq_len = x.shape
    d = token_table.shape[1]
    xi = x.astype(jnp.int32)
    span = seq_len // NUM_WORKERS

    mesh = plsc.VectorSubcoreMesh(core_axis_name="c", subcore_axis_name="s")
    run = pl.kernel(
        functools.partial(_embed_body, seq_len),
        out_type=jax.ShapeDtypeStruct((batch, seq_len, d), jnp.float32),
        mesh=mesh,
        scratch_types=[
            pltpu.VMEM((NBATCH, span), jnp.int32),
            pltpu.VMEM((NBATCH, NSLOT, CHUNK, d), jnp.float32),
            pltpu.VMEM((2, CHUNK, d), jnp.float32),
            pltpu.SemaphoreType.DMA((NBATCH, NSLOT)),
            pltpu.SemaphoreType.DMA((NBATCH, NSLOT)),
            pltpu.SemaphoreType.DMA((2,)),
        ],
    )
    return run(token_table, pos_table, xi)
